# baseline (speedup 1.0000x reference)
import functools

import jax
import jax.numpy as jnp
from jax import lax
from jax.experimental import pallas as pl
from jax.experimental.pallas import tpu as pltpu
from jax.experimental.pallas import tpu_sc as plsc

VOCAB = 1000000
DIM = 16
BATCH = 4096
CTX_LEN = 200
Q_LEN = 20

NCTX = BATCH * CTX_LEN
NQ = BATCH * Q_LEN
NW = 32
CTX_PW = NCTX // NW
Q_PW = NQ // NW
CH = 2560
CTX_CHUNKS = CTX_PW // CH

_mesh = plsc.VectorSubcoreMesh(core_axis_name="c", subcore_axis_name="s")


@functools.partial(
    pl.kernel,
    mesh=_mesh,
    out_type=[
        jax.ShapeDtypeStruct((3 * NCTX, DIM), jnp.float32),
        jax.ShapeDtypeStruct((3 * NQ, DIM), jnp.float32),
    ],
    scratch_types=[
        pltpu.VMEM((CH,), jnp.int32),
        pltpu.VMEM((CH, DIM), jnp.float32),
        pltpu.SemaphoreType.DMA,
    ],
    compiler_params=pltpu.CompilerParams(use_tc_tiling_on_sc=False),
)
def _embed(w0, w1, w2, ctx, q, out_ctx, out_q, idx_v, rows_v, sem):
    wid = lax.axis_index("s") * 2 + lax.axis_index("c")

    for t, w in enumerate((w0, w1, w2)):
        def ctx_body(j, carry, w=w, t=t):
            off = wid * CTX_PW + j * CH
            pltpu.sync_copy(ctx.at[pl.ds(off, CH)], idx_v)
            pltpu.async_copy(w.at[idx_v], rows_v, sem).wait()
            pltpu.sync_copy(rows_v, out_ctx.at[pl.ds(t * NCTX + off, CH)])
            return carry

        lax.fori_loop(0, CTX_CHUNKS, ctx_body, 0)

        qoff = wid * Q_PW
        pltpu.sync_copy(q.at[pl.ds(qoff, Q_PW)], idx_v)
        pltpu.async_copy(w.at[idx_v], rows_v, sem).wait()
        pltpu.sync_copy(rows_v, out_q.at[pl.ds(t * NQ + qoff, Q_PW)])


def kernel(W_word, W_pos, W_kg, context, question):
    ctx = context.reshape(-1)
    q = question.reshape(-1)
    out_ctx, out_q = _embed(W_word, W_pos, W_kg, ctx, q)
    return (
        out_ctx.reshape(3 * BATCH, CTX_LEN, DIM),
        out_q.reshape(3 * BATCH, Q_LEN, DIM),
    )


# speedup vs baseline: 1.3189x; 1.3189x over previous
import functools

import jax
import jax.numpy as jnp
from jax import lax
from jax.experimental import pallas as pl
from jax.experimental.pallas import tpu as pltpu
from jax.experimental.pallas import tpu_sc as plsc

VOCAB = 1000000
DIM = 16
BATCH = 4096
CTX_LEN = 200
Q_LEN = 20

NW = 32
CH = 2048
NBT = CH // 128
HALF = CH * 8

OC_LEN = CTX_LEN * 2 * 96 * 8 * 128
OQ_LEN = Q_LEN * 2 * 96 * 8 * 128

_mesh = plsc.VectorSubcoreMesh(core_axis_name="c", subcore_axis_name="s")


@functools.partial(
    pl.kernel,
    mesh=_mesh,
    out_type=[
        jax.ShapeDtypeStruct((OC_LEN,), jnp.float32),
        jax.ShapeDtypeStruct((OQ_LEN,), jnp.float32),
    ],
    scratch_types=[
        pltpu.VMEM((CH,), jnp.int32),
        pltpu.VMEM((CH, DIM), jnp.float32),
        pltpu.VMEM((HALF,), jnp.float32),
        pltpu.VMEM((HALF,), jnp.float32),
        pltpu.SemaphoreType.DMA,
    ],
    compiler_params=pltpu.CompilerParams(
        use_tc_tiling_on_sc=False, needs_layout_passes=False
    ),
)
def _embed(w0, w1, w2, ctxf, qf, oc, oq, idx_v, rows_v, tr0, tr1, sem):
    wid = lax.axis_index("s") * 2 + lax.axis_index("c")
    lane = lax.iota(jnp.int32, 16)
    pat = lane * 128
    m_lo = lane < 8
    m_hi = lane >= 8

    def do_half(w, src, dst, s, t, h):
        pltpu.sync_copy(src.at[pl.ds(s * BATCH + h * CH, CH)], idx_v)
        pltpu.async_copy(w.at[idx_v], rows_v, sem).wait()

        def bt_body(bt, carry):
            vbase0 = pat + bt * 1024
            vbase1 = vbase0 - 1024

            def bb_body(jj, carry2):
                for k in range(8):
                    i = bt * 128 + jj * 8 + k
                    row = rows_v[i, :]
                    a0 = vbase0 + (jj * 8 + k)
                    a1 = vbase1 + (jj * 8 + k)
                    plsc.store_scatter(tr0, [a0], row, mask=m_lo)
                    plsc.store_scatter(tr1, [a1], row, mask=m_hi)
                return carry2

            lax.fori_loop(0, 16, bb_body, 0)
            return carry

        lax.fori_loop(0, NBT, bt_body, 0)

        c0 = t * 32 + h * NBT
        pltpu.sync_copy(tr0, dst.at[pl.ds(((s * 2 + 0) * 96 + c0) * 1024, HALF)])
        pltpu.sync_copy(tr1, dst.at[pl.ds(((s * 2 + 1) * 96 + c0) * 1024, HALF)])

    for t, w in enumerate((w0, w1, w2)):
        r0 = (wid + 11 * t) % 32
        n_s = jnp.where(r0 < CTX_LEN % NW, CTX_LEN // NW + 1, CTX_LEN // NW)

        def ctx_body(j, carry, w=w, t=t, r0=r0):
            s = r0 + NW * j
            do_half(w, ctxf, oc, s, t, 0)
            do_half(w, ctxf, oc, s, t, 1)
            return carry

        lax.fori_loop(0, n_s, ctx_body, 0)

        rq = (wid + 7 * t) % 32

        @pl.when(rq < Q_LEN)
        def _(w=w, t=t, rq=rq):
            do_half(w, qf, oq, rq, t, 0)
            do_half(w, qf, oq, rq, t, 1)


def kernel(W_word, W_pos, W_kg, context, question):
    ctxf = context.T.reshape(-1)
    qf = question.T.reshape(-1)
    oc, oq = _embed(W_word, W_pos, W_kg, ctxf, qf)
    out_c = (
        oc.reshape(CTX_LEN, 2, 96, 8, 128)
        .transpose(2, 4, 0, 1, 3)
        .reshape(3 * BATCH, CTX_LEN, DIM)
    )
    out_q = (
        oq.reshape(Q_LEN, 2, 96, 8, 128)
        .transpose(2, 4, 0, 1, 3)
        .reshape(3 * BATCH, Q_LEN, DIM)
    )
    return out_c, out_q


# speedup vs baseline: 1.4416x; 1.0930x over previous
import functools

import jax
import jax.numpy as jnp
from jax import lax
from jax.experimental import pallas as pl
from jax.experimental.pallas import tpu as pltpu
from jax.experimental.pallas import tpu_sc as plsc

VOCAB = 1000000
DIM = 16
BATCH = 4096
CTX_LEN = 200
Q_LEN = 20

NW = 32
CH = 1024
NBT = CH // 128
PLANE = NBT * 1024

OC_LEN = CTX_LEN * 2 * 96 * 8 * 128
OQ_LEN = Q_LEN * 2 * 96 * 8 * 128

_mesh = plsc.VectorSubcoreMesh(core_axis_name="c", subcore_axis_name="s")


@functools.partial(
    pl.kernel,
    mesh=_mesh,
    out_type=[
        jax.ShapeDtypeStruct((OC_LEN,), jnp.float32),
        jax.ShapeDtypeStruct((OQ_LEN,), jnp.float32),
    ],
    scratch_types=[
        pltpu.VMEM((BATCH,), jnp.int32),
        pltpu.VMEM((BATCH,), jnp.int32),
        [pltpu.VMEM((CH, DIM), jnp.float32) for _ in range(4)],
        [pltpu.VMEM((PLANE,), jnp.float32) for _ in range(4)],
        [pltpu.SemaphoreType.DMA for _ in range(4)],
        [pltpu.SemaphoreType.DMA for _ in range(2)],
        [pltpu.SemaphoreType.DMA for _ in range(2)],
    ],
    compiler_params=pltpu.CompilerParams(
        use_tc_tiling_on_sc=False, needs_layout_passes=False
    ),
)
def _embed(w0, w1, w2, ctxf, qf, oc, oq, idxA, idxB, rows, trs, gsem, isem, wsem):
    wid = lax.axis_index("s") * 2 + lax.axis_index("c")
    lane = lax.iota(jnp.int32, 16)
    pat = lane * 128
    m_lo = lane < 8
    m_hi = lane >= 8

    def transpose_chunk(rv, p0, p1):
        def bt_body(bt, carry):
            vbase0 = pat + bt * 1024
            vbase1 = vbase0 - 1024

            def bb_body(jj, carry2):
                for k in range(8):
                    i = bt * 128 + jj * 8 + k
                    row = rv[i, :]
                    bb = jj * 8 + k
                    plsc.store_scatter(p0, [vbase0 + bb], row, mask=m_lo)
                    plsc.store_scatter(p1, [vbase1 + bb], row, mask=m_hi)
                return carry2

            lax.fori_loop(0, 16, bb_body, 0)
            return carry

        lax.fori_loop(0, NBT, bt_body, 0)

    def fire_gather(w, idx_ref, b):
        pltpu.async_copy(w.at[idx_ref.at[pl.ds(b * CH, CH)]], rows[b], gsem[b])

    def wait_gather(w, b):
        pltpu.make_async_copy(w.at[idxA.at[pl.ds(0, CH)]], rows[b], gsem[b]).wait()

    def fire_writes(dst, s, t, b, pr):
        c0 = t * 32 + b * NBT
        dst0 = dst.at[pl.ds(((s * 2 + 0) * 96 + c0) * 1024, PLANE)]
        dst1 = dst.at[pl.ds(((s * 2 + 1) * 96 + c0) * 1024, PLANE)]
        pltpu.async_copy(trs[2 * pr], dst0, wsem[pr])
        pltpu.async_copy(trs[2 * pr + 1], dst1, wsem[pr])

    def wait_writes(dst, pr):
        pltpu.make_async_copy(trs[2 * pr], dst.at[pl.ds(0, PLANE)], wsem[pr]).wait()
        pltpu.make_async_copy(trs[2 * pr + 1], dst.at[pl.ds(0, PLANE)], wsem[pr]).wait()

    def fire_idx(src, s, idx_ref, pr):
        pltpu.async_copy(src.at[pl.ds(s * BATCH, BATCH)], idx_ref, isem[pr])

    def wait_idx(src, idx_ref, pr):
        pltpu.make_async_copy(src.at[pl.ds(0, BATCH)], idx_ref, isem[pr]).wait()

    for t, w in enumerate((w0, w1, w2)):
        r0 = (wid + 11 * t) % 32

        pltpu.sync_copy(ctxf.at[pl.ds(r0 * BATCH, BATCH)], idxA)
        for b in range(4):
            fire_gather(w, idxA, b)
        fire_idx(ctxf, r0 + NW, idxB, 1)

        def section(g, idx_cur, pr_cur, idx_nxt, pr_nxt, w=w, t=t, r0=r0):
            s = r0 + NW * g

            @pl.when(s < CTX_LEN)
            def _():
                s_n1 = s + NW
                s_n2 = s + 2 * NW
                for b in range(4):
                    wait_gather(w, b)
                    if b < 2:
                        @pl.when(g > 0)
                        def _():
                            wait_writes(oc, b % 2)
                    else:
                        wait_writes(oc, b % 2)
                    transpose_chunk(rows[b], trs[2 * (b % 2)], trs[2 * (b % 2) + 1])
                    fire_writes(oc, s, t, b, b % 2)
                    if b == 0:
                        @pl.when(s_n1 < CTX_LEN)
                        def _():
                            wait_idx(ctxf, idx_nxt, pr_nxt)
                    if b == 3:
                        @pl.when(s_n2 < CTX_LEN)
                        def _():
                            fire_idx(ctxf, s_n2, idx_cur, pr_cur)

                    @pl.when(s_n1 < CTX_LEN)
                    def _(b=b):
                        fire_gather(w, idx_nxt, b)

        def pair_body(jj, carry):
            section(2 * jj, idxA, 0, idxB, 1)
            section(2 * jj + 1, idxB, 1, idxA, 0)
            return carry

        lax.fori_loop(0, 4, pair_body, 0)
        wait_writes(oc, 0)
        wait_writes(oc, 1)

        rq = (wid + 7 * t) % 32

        @pl.when(rq < Q_LEN)
        def _(w=w, t=t, rq=rq):
            pltpu.sync_copy(qf.at[pl.ds(rq * BATCH, BATCH)], idxA)
            for b in range(4):
                fire_gather(w, idxA, b)
            for b in range(4):
                wait_gather(w, b)
                if b >= 2:
                    wait_writes(oq, b % 2)
                transpose_chunk(rows[b], trs[2 * (b % 2)], trs[2 * (b % 2) + 1])
                fire_writes(oq, rq, t, b, b % 2)
            wait_writes(oq, 0)
            wait_writes(oq, 1)


def kernel(W_word, W_pos, W_kg, context, question):
    ctxf = context.T.reshape(-1)
    qf = question.T.reshape(-1)
    oc, oq = _embed(W_word, W_pos, W_kg, ctxf, qf)
    out_c = (
        oc.reshape(CTX_LEN, 2, 96, 8, 128)
        .transpose(2, 4, 0, 1, 3)
        .reshape(3 * BATCH, CTX_LEN, DIM)
    )
    out_q = (
        oq.reshape(Q_LEN, 2, 96, 8, 128)
        .transpose(2, 4, 0, 1, 3)
        .reshape(3 * BATCH, Q_LEN, DIM)
    )
    return out_c, out_q
